# baseline (speedup 1.0000x reference)
import functools

import jax
import jax.numpy as jnp
from jax import lax
from jax.experimental import pallas as pl
from jax.experimental.pallas import tpu as pltpu
from jax.experimental.pallas import tpu_sc as plsc

N = 100000
E = 6400000
NC = 2
NS = 16
NW = NC * NS
NP = 101888
SUB = NP // NS
EW = E // NW
CH = 2000
ITERS = EW // CH


def _mesh():
    return plsc.VectorSubcoreMesh(core_axis_name="c", subcore_axis_name="s")


def _hist(dst, zeros1, ones1):
    @functools.partial(
        pl.kernel,
        out_type=jax.ShapeDtypeStruct((NC, NP), jnp.float32),
        mesh=_mesh(),
        compiler_params=pltpu.CompilerParams(use_tc_tiling_on_sc=False),
        scratch_types=[
            pltpu.VMEM((CH,), jnp.int32),
            pltpu.VMEM((CH,), jnp.float32),
            pltpu.VMEM((SUB,), jnp.float32),
            pltpu.VMEM_SHARED((NP,), jnp.float32),
        ],
    )
    def k(dst_hbm, z_hbm, ones_hbm, out_hbm, idx_v, ones_v, buf_v, acc_sh):
        cid = lax.axis_index("c")
        sid = lax.axis_index("s")
        wid = cid * NS + sid
        pltpu.sync_copy(ones_hbm, ones_v)
        pltpu.sync_copy(z_hbm.at[pl.ds(sid * SUB, SUB)], buf_v)
        pltpu.sync_copy(buf_v, acc_sh.at[pl.ds(sid * SUB, SUB)])
        plsc.subcore_barrier()

        def body(i, _):
            base = wid * EW + i * CH
            pltpu.sync_copy(dst_hbm.at[pl.ds(base, CH)], idx_v)
            pltpu.sync_copy(ones_v, acc_sh.at[idx_v], add=True)
            return 0

        lax.fori_loop(0, ITERS, body, 0)
        plsc.subcore_barrier()
        pltpu.sync_copy(acc_sh.at[pl.ds(sid * SUB, SUB)], buf_v)
        pltpu.sync_copy(buf_v, out_hbm.at[cid, pl.ds(sid * SUB, SUB)])

    return k(dst, zeros1, ones1)


def _msgpass(src, dst, tpk, zeros1):
    @functools.partial(
        pl.kernel,
        out_type=jax.ShapeDtypeStruct((NC, 2, NP), jnp.float32),
        mesh=_mesh(),
        compiler_params=pltpu.CompilerParams(
            use_tc_tiling_on_sc=False, needs_layout_passes=False),
        scratch_types=[
            pltpu.VMEM((CH,), jnp.int32),
            pltpu.VMEM((CH,), jnp.int32),
            pltpu.VMEM((CH,), jnp.float32),
            pltpu.VMEM((CH,), jnp.float32),
            pltpu.VMEM((SUB,), jnp.float32),
            pltpu.VMEM((NP,), jnp.int32),
            pltpu.VMEM_SHARED((NP,), jnp.float32),
            pltpu.VMEM_SHARED((NP,), jnp.float32),
        ],
    )
    def k(src_hbm, dst_hbm, tpk_hbm, z_hbm, out_hbm, idxs_v, idxd_v,
          m0_v, m1_v, buf_v, tab_v, a0_sh, a1_sh):
        cid = lax.axis_index("c")
        sid = lax.axis_index("s")
        wid = cid * NS + sid
        sl = pl.ds(sid * SUB, SUB)
        pltpu.sync_copy(tpk_hbm, tab_v)
        pltpu.sync_copy(z_hbm.at[sl], buf_v)
        pltpu.sync_copy(buf_v, a0_sh.at[sl])
        pltpu.sync_copy(buf_v, a1_sh.at[sl])
        plsc.subcore_barrier()

        mask_hi = jnp.full((16,), -65536, dtype=jnp.int32)

        def unpack_body(j, _):
            idx = idxs_v[pl.ds(j * 16, 16)]
            w = plsc.load_gather(tab_v, [idx])
            m0_v[pl.ds(j * 16, 16)] = plsc.bitcast(
                lax.shift_left(w, 16), jnp.float32)
            m1_v[pl.ds(j * 16, 16)] = plsc.bitcast(
                jnp.bitwise_and(w, mask_hi), jnp.float32)
            return 0

        def body(i, _):
            base = wid * EW + i * CH
            pltpu.sync_copy(src_hbm.at[pl.ds(base, CH)], idxs_v)
            pltpu.sync_copy(dst_hbm.at[pl.ds(base, CH)], idxd_v)
            lax.fori_loop(0, CH // 16, unpack_body, 0)
            pltpu.sync_copy(m0_v, a0_sh.at[idxd_v], add=True)
            pltpu.sync_copy(m1_v, a1_sh.at[idxd_v], add=True)
            return 0

        lax.fori_loop(0, ITERS, body, 0)
        plsc.subcore_barrier()
        pltpu.sync_copy(a0_sh.at[sl], buf_v)
        pltpu.sync_copy(buf_v, out_hbm.at[cid, 0, sl])
        pltpu.sync_copy(a1_sh.at[sl], buf_v)
        pltpu.sync_copy(buf_v, out_hbm.at[cid, 1, sl])

    return k(src, dst, tpk, zeros1)


def _pack_bf16(g0, g1):
    b0 = lax.bitcast_convert_type(g0.astype(jnp.bfloat16), jnp.uint16)
    b1 = lax.bitcast_convert_type(g1.astype(jnp.bfloat16), jnp.uint16)
    w = (b1.astype(jnp.uint32) << 16) | b0.astype(jnp.uint32)
    return lax.bitcast_convert_type(w, jnp.int32)


def _tc_pre(degp, xT, W1):
    def body(degp_ref, xT_ref, w_ref, dinv_ref, gT_ref, pk_ref):
        deg = degp_ref[0:1, :] + degp_ref[1:2, :] + 1.0
        dinv = lax.rsqrt(deg)
        dinv_ref[...] = dinv
        x0 = xT_ref[0:1, :]
        x1 = xT_ref[1:2, :]
        g0 = dinv * (x0 * w_ref[0:1, 0:1] + x1 * w_ref[1:2, 0:1])
        g1 = dinv * (x0 * w_ref[0:1, 1:2] + x1 * w_ref[1:2, 1:2])
        gT_ref[...] = jnp.concatenate([g0, g1], axis=0)
        pk_ref[...] = _pack_bf16(g0, g1)

    return pl.pallas_call(
        body,
        out_shape=[
            jax.ShapeDtypeStruct((1, NP), jnp.float32),
            jax.ShapeDtypeStruct((2, NP), jnp.float32),
            jax.ShapeDtypeStruct((1, NP), jnp.int32),
        ],
    )(degp, xT, W1)


def _tc_mid(Sa, Sb, gT, dinv, W2, b1):
    def body(sa_ref, sb_ref, gT_ref, dinv_ref, w_ref, b_ref, g2T_ref, pk_ref):
        dinv = dinv_ref[...]
        o = dinv * (sa_ref[...] + sb_ref[...] + gT_ref[...]) + b_ref[...]
        o0 = o[0:1, :]
        o1 = o[1:2, :]
        g0 = dinv * (o0 * w_ref[0:1, 0:1] + o1 * w_ref[1:2, 0:1])
        g1 = dinv * (o0 * w_ref[0:1, 1:2] + o1 * w_ref[1:2, 1:2])
        g2T_ref[...] = jnp.concatenate([g0, g1], axis=0)
        pk_ref[...] = _pack_bf16(g0, g1)

    return pl.pallas_call(
        body,
        out_shape=[
            jax.ShapeDtypeStruct((2, NP), jnp.float32),
            jax.ShapeDtypeStruct((1, NP), jnp.int32),
        ],
    )(Sa, Sb, gT, dinv, W2, b1)


def _tc_post(Sa, Sb, gT, dinv, b2):
    def body(sa_ref, sb_ref, gT_ref, dinv_ref, b_ref, out_ref):
        out_ref[...] = (
            dinv_ref[...] * (sa_ref[...] + sb_ref[...] + gT_ref[...])
            + b_ref[...]
        )

    return pl.pallas_call(
        body,
        out_shape=jax.ShapeDtypeStruct((2, NP), jnp.float32),
    )(Sa, Sb, gT, dinv, b2)


def kernel(x, edge_index, W1, b1, W2, b2):
    assert x.shape == (N, 2) and edge_index.shape == (2, E)
    src = edge_index[0]
    dst = edge_index[1]
    xT = jnp.pad(x, ((0, NP - N), (0, 0))).T
    b1c = jnp.broadcast_to(b1.reshape(2, 1), (2, NP))
    b2c = jnp.broadcast_to(b2.reshape(2, 1), (2, NP))
    zeros1 = jnp.zeros((NP,), jnp.float32)
    ones1 = jnp.ones((CH,), jnp.float32)

    degp = _hist(dst, zeros1, ones1)
    dinv, g1T, tpk1 = _tc_pre(degp, xT, W1)

    S1p = _msgpass(src, dst, tpk1.reshape(NP), zeros1)
    g2T, tpk2 = _tc_mid(S1p[0], S1p[1], g1T, dinv, W2, b1c)

    S2p = _msgpass(src, dst, tpk2.reshape(NP), zeros1)
    out2T = _tc_post(S2p[0], S2p[1], g2T, dinv, b2c)

    return out2T.T[:N]


# speedup vs baseline: 1.2800x; 1.2800x over previous
import functools

import jax
import jax.numpy as jnp
from jax import lax
from jax.experimental import pallas as pl
from jax.experimental.pallas import tpu as pltpu
from jax.experimental.pallas import tpu_sc as plsc

N = 100000
E = 6400000
NC = 2
NS = 16
NW = NC * NS
NP = 101888
SUB = NP // NS
EW = E // NW
CH = 8000
ITERS = EW // CH


def _mesh():
    return plsc.VectorSubcoreMesh(core_axis_name="c", subcore_axis_name="s")


def _hist(dst, zeros1, ones1):
    @functools.partial(
        pl.kernel,
        out_type=jax.ShapeDtypeStruct((NC, NP), jnp.float32),
        mesh=_mesh(),
        compiler_params=pltpu.CompilerParams(use_tc_tiling_on_sc=False),
        scratch_types=[
            pltpu.VMEM((CH,), jnp.int32),
            pltpu.VMEM((CH,), jnp.int32),
            pltpu.VMEM((CH,), jnp.float32),
            pltpu.VMEM((SUB,), jnp.float32),
            pltpu.VMEM_SHARED((NP,), jnp.float32),
            pltpu.SemaphoreType.DMA,
            pltpu.SemaphoreType.DMA,
        ],
    )
    def k(dst_hbm, z_hbm, ones_hbm, out_hbm, idx_v, idx2_v, ones_v, buf_v,
          acc_sh, sem_i, sem_a):
        cid = lax.axis_index("c")
        sid = lax.axis_index("s")
        wid = cid * NS + sid
        pltpu.sync_copy(ones_hbm, ones_v)
        pltpu.sync_copy(z_hbm.at[pl.ds(sid * SUB, SUB)], buf_v)
        pltpu.sync_copy(buf_v, acc_sh.at[pl.ds(sid * SUB, SUB)])
        plsc.subcore_barrier()

        base0 = wid * EW
        pltpu.async_copy(dst_hbm.at[pl.ds(base0, CH)], idx_v, sem_i).wait()

        def body(i, _):
            base = wid * EW + i * CH

            @pl.when(i % 2 == 0)
            def _():
                a = pltpu.async_copy(ones_v, acc_sh.at[idx_v], sem_a,
                                     add=True)
                ld = pltpu.async_copy(dst_hbm.at[pl.ds(base + CH, CH)],
                                      idx2_v, sem_i)
                a.wait()
                ld.wait()

            @pl.when(i % 2 == 1)
            def _():
                a = pltpu.async_copy(ones_v, acc_sh.at[idx2_v], sem_a,
                                     add=True)
                ld = pltpu.async_copy(dst_hbm.at[pl.ds(base + CH, CH)],
                                      idx_v, sem_i)
                a.wait()
                ld.wait()

            return 0

        lax.fori_loop(0, ITERS - 1, body, 0)
        last_idx = idx_v if (ITERS - 1) % 2 == 0 else idx2_v
        pltpu.async_copy(ones_v, acc_sh.at[last_idx], sem_a, add=True).wait()
        plsc.subcore_barrier()
        pltpu.sync_copy(acc_sh.at[pl.ds(sid * SUB, SUB)], buf_v)
        pltpu.sync_copy(buf_v, out_hbm.at[cid, pl.ds(sid * SUB, SUB)])

    return k(dst, zeros1, ones1)


def _msgpass(src, dst, t0, t1, zeros1):
    @functools.partial(
        pl.kernel,
        out_type=jax.ShapeDtypeStruct((NC, 2, NP), jnp.float32),
        mesh=_mesh(),
        compiler_params=pltpu.CompilerParams(use_tc_tiling_on_sc=False),
        scratch_types=[
            pltpu.VMEM((CH,), jnp.int32),
            pltpu.VMEM((CH,), jnp.int32),
            pltpu.VMEM((CH,), jnp.float32),
            pltpu.VMEM((CH,), jnp.float32),
            pltpu.VMEM((SUB,), jnp.float32),
            pltpu.VMEM_SHARED((NP,), jnp.float32),
            pltpu.VMEM_SHARED((NP,), jnp.float32),
            pltpu.VMEM_SHARED((NP,), jnp.float32),
            pltpu.VMEM_SHARED((NP,), jnp.float32),
            pltpu.SemaphoreType.DMA,
            pltpu.SemaphoreType.DMA,
            pltpu.SemaphoreType.DMA,
            pltpu.SemaphoreType.DMA,
        ],
    )
    def k(src_hbm, dst_hbm, t0_hbm, t1_hbm, z_hbm, out_hbm, idxs_v, idxd_v,
          m0_v, m1_v, buf_v, t0_sh, t1_sh, a0_sh, a1_sh,
          sem_i, sem_j, sem_g, sem_a):
        cid = lax.axis_index("c")
        sid = lax.axis_index("s")
        wid = cid * NS + sid
        sl = pl.ds(sid * SUB, SUB)
        pltpu.sync_copy(z_hbm.at[sl], buf_v)
        pltpu.sync_copy(buf_v, a0_sh.at[sl])
        pltpu.sync_copy(buf_v, a1_sh.at[sl])
        pltpu.sync_copy(t0_hbm.at[sl], buf_v)
        pltpu.sync_copy(buf_v, t0_sh.at[sl])
        pltpu.sync_copy(t1_hbm.at[sl], buf_v)
        pltpu.sync_copy(buf_v, t1_sh.at[sl])
        plsc.subcore_barrier()

        def body(i, _):
            base = wid * EW + i * CH
            li = pltpu.async_copy(src_hbm.at[pl.ds(base, CH)], idxs_v, sem_i)
            lj = pltpu.async_copy(dst_hbm.at[pl.ds(base, CH)], idxd_v, sem_j)
            li.wait()
            g0 = pltpu.async_copy(t0_sh.at[idxs_v], m0_v, sem_g)
            g1 = pltpu.async_copy(t1_sh.at[idxs_v], m1_v, sem_g)
            lj.wait()
            g0.wait()
            g1.wait()
            a0 = pltpu.async_copy(m0_v, a0_sh.at[idxd_v], sem_a, add=True)
            a1 = pltpu.async_copy(m1_v, a1_sh.at[idxd_v], sem_a, add=True)
            a0.wait()
            a1.wait()
            return 0

        lax.fori_loop(0, ITERS, body, 0)
        plsc.subcore_barrier()
        pltpu.sync_copy(a0_sh.at[sl], buf_v)
        pltpu.sync_copy(buf_v, out_hbm.at[cid, 0, sl])
        pltpu.sync_copy(a1_sh.at[sl], buf_v)
        pltpu.sync_copy(buf_v, out_hbm.at[cid, 1, sl])

    return k(src, dst, t0, t1, zeros1)


def _tc_pre(degp, xT, W1):
    def body(degp_ref, xT_ref, w_ref, dinv_ref, gT_ref):
        deg = degp_ref[0:1, :] + degp_ref[1:2, :] + 1.0
        dinv = lax.rsqrt(deg)
        dinv_ref[...] = dinv
        x0 = xT_ref[0:1, :]
        x1 = xT_ref[1:2, :]
        g0 = dinv * (x0 * w_ref[0:1, 0:1] + x1 * w_ref[1:2, 0:1])
        g1 = dinv * (x0 * w_ref[0:1, 1:2] + x1 * w_ref[1:2, 1:2])
        gT_ref[...] = jnp.concatenate([g0, g1], axis=0)

    return pl.pallas_call(
        body,
        out_shape=[
            jax.ShapeDtypeStruct((1, NP), jnp.float32),
            jax.ShapeDtypeStruct((2, NP), jnp.float32),
        ],
    )(degp, xT, W1)


def _tc_mid(Sa, Sb, gT, dinv, W2, b1):
    def body(sa_ref, sb_ref, gT_ref, dinv_ref, w_ref, b_ref, g2T_ref):
        dinv = dinv_ref[...]
        o = dinv * (sa_ref[...] + sb_ref[...] + gT_ref[...]) + b_ref[...]
        o0 = o[0:1, :]
        o1 = o[1:2, :]
        g0 = dinv * (o0 * w_ref[0:1, 0:1] + o1 * w_ref[1:2, 0:1])
        g1 = dinv * (o0 * w_ref[0:1, 1:2] + o1 * w_ref[1:2, 1:2])
        g2T_ref[...] = jnp.concatenate([g0, g1], axis=0)

    return pl.pallas_call(
        body,
        out_shape=jax.ShapeDtypeStruct((2, NP), jnp.float32),
    )(Sa, Sb, gT, dinv, W2, b1)


def _tc_post(Sa, Sb, gT, dinv, b2):
    def body(sa_ref, sb_ref, gT_ref, dinv_ref, b_ref, out_ref):
        out_ref[...] = (
            dinv_ref[...] * (sa_ref[...] + sb_ref[...] + gT_ref[...])
            + b_ref[...]
        )

    return pl.pallas_call(
        body,
        out_shape=jax.ShapeDtypeStruct((2, NP), jnp.float32),
    )(Sa, Sb, gT, dinv, b2)


def kernel(x, edge_index, W1, b1, W2, b2):
    assert x.shape == (N, 2) and edge_index.shape == (2, E)
    src = edge_index[0]
    dst = edge_index[1]
    xT = jnp.pad(x, ((0, NP - N), (0, 0))).T
    b1c = jnp.broadcast_to(b1.reshape(2, 1), (2, NP))
    b2c = jnp.broadcast_to(b2.reshape(2, 1), (2, NP))
    zeros1 = jnp.zeros((NP,), jnp.float32)
    ones1 = jnp.ones((CH,), jnp.float32)

    degp = _hist(dst, zeros1, ones1)
    dinv, g1T = _tc_pre(degp, xT, W1)

    S1p = _msgpass(src, dst, g1T[0], g1T[1], zeros1)
    g2T = _tc_mid(S1p[0], S1p[1], g1T, dinv, W2, b1c)

    S2p = _msgpass(src, dst, g2T[0], g2T[1], zeros1)
    out2T = _tc_post(S2p[0], S2p[1], g2T, dinv, b2c)

    return out2T.T[:N]


# speedup vs baseline: 1.7962x; 1.4033x over previous
import functools

import jax
import jax.numpy as jnp
from jax import lax
from jax.experimental import pallas as pl
from jax.experimental.pallas import tpu as pltpu
from jax.experimental.pallas import tpu_sc as plsc

N = 100000
E = 6400000
NC = 2
NS = 16
NW = NC * NS
NP = 101888
SUB = NP // NS
EW = E // NW
CH = 8000
ITERS = EW // CH
MCH = 2000
MITERS = EW // MCH
QS = SUB // 4
VCH = ((0, 1984), (1984, 1984), (3968, 1984), (5952, 416))


def _mesh():
    return plsc.VectorSubcoreMesh(core_axis_name="c", subcore_axis_name="s")


def _rsqrt16(x):
    i = plsc.bitcast(x, jnp.int32)
    y = plsc.bitcast(jnp.int32(0x5F3759DF) - lax.shift_right_logical(i, 1),
                     jnp.float32)
    for _ in range(3):
        y = y * (1.5 - 0.5 * x * y * y)
    return y


def _pack16(g0, g1):
    b0 = plsc.bitcast(g0, jnp.int32)
    b1 = plsc.bitcast(g1, jnp.int32)
    r0 = lax.shift_right_logical(
        b0 + 0x7FFF + jnp.bitwise_and(lax.shift_right_logical(b0, 16), 1), 16)
    r1 = lax.shift_right_logical(
        b1 + 0x7FFF + jnp.bitwise_and(lax.shift_right_logical(b1, 16), 1), 16)
    return jnp.bitwise_or(lax.shift_left(r1, 16), r0)


def _hist(dst, zeros1, ones1):
    @functools.partial(
        pl.kernel,
        out_type=jax.ShapeDtypeStruct((NC, NP), jnp.float32),
        mesh=_mesh(),
        compiler_params=pltpu.CompilerParams(use_tc_tiling_on_sc=False),
        scratch_types=[
            pltpu.VMEM((CH,), jnp.int32),
            pltpu.VMEM((CH,), jnp.int32),
            pltpu.VMEM((CH,), jnp.float32),
            pltpu.VMEM((SUB,), jnp.float32),
            pltpu.VMEM_SHARED((NP,), jnp.float32),
            pltpu.SemaphoreType.DMA,
            pltpu.SemaphoreType.DMA,
        ],
    )
    def k(dst_hbm, z_hbm, ones_hbm, out_hbm, idx_v, idx2_v, ones_v, buf_v,
          acc_sh, sem_i, sem_a):
        cid = lax.axis_index("c")
        sid = lax.axis_index("s")
        wid = cid * NS + sid
        pltpu.sync_copy(ones_hbm, ones_v)
        pltpu.sync_copy(z_hbm.at[pl.ds(sid * SUB, SUB)], buf_v)
        pltpu.sync_copy(buf_v, acc_sh.at[pl.ds(sid * SUB, SUB)])
        plsc.subcore_barrier()

        base0 = wid * EW
        pltpu.async_copy(dst_hbm.at[pl.ds(base0, CH)], idx_v, sem_i).wait()

        def body(i, _):
            base = wid * EW + i * CH

            @pl.when(i % 2 == 0)
            def _():
                a = pltpu.async_copy(ones_v, acc_sh.at[idx_v], sem_a,
                                     add=True)
                ld = pltpu.async_copy(dst_hbm.at[pl.ds(base + CH, CH)],
                                      idx2_v, sem_i)
                a.wait()
                ld.wait()

            @pl.when(i % 2 == 1)
            def _():
                a = pltpu.async_copy(ones_v, acc_sh.at[idx2_v], sem_a,
                                     add=True)
                ld = pltpu.async_copy(dst_hbm.at[pl.ds(base + CH, CH)],
                                      idx_v, sem_i)
                a.wait()
                ld.wait()

            return 0

        lax.fori_loop(0, ITERS - 1, body, 0)
        last_idx = idx_v if (ITERS - 1) % 2 == 0 else idx2_v
        pltpu.async_copy(ones_v, acc_sh.at[last_idx], sem_a, add=True).wait()
        plsc.subcore_barrier()
        pltpu.sync_copy(acc_sh.at[pl.ds(sid * SUB, SUB)], buf_v)
        pltpu.sync_copy(buf_v, out_hbm.at[cid, pl.ds(sid * SUB, SUB)])

    return k(dst, zeros1, ones1)


def _msg_body(src_hbm, dst_hbm, z_hbm, out_hbm,
              is0, is1, id0, id1, m00, m01, m10, m11, tab_v, a0_sh, a1_sh,
              sem_i, sem_s, cid, sid, wid):
    idx_s = (is0, is1)
    idx_d = (id0, id1)
    m0 = (m00, m01)
    m1 = (m10, m11)
    for c in range(4):
        qsl = pl.ds(sid * SUB + c * QS, QS)
        pltpu.sync_copy(z_hbm.at[pl.ds(c * QS, QS)], m00.at[pl.ds(0, QS)])
        pltpu.sync_copy(m00.at[pl.ds(0, QS)], a0_sh.at[qsl])
        pltpu.sync_copy(m00.at[pl.ds(0, QS)], a1_sh.at[qsl])
    plsc.subcore_barrier()

    mask_hi = jnp.full((16,), -65536, dtype=jnp.int32)

    def make_unpack(b):
        def unpack_body(j, _):
            for u in range(5):
                off = j * 80 + u * 16
                idx = idx_s[b][pl.ds(off, 16)]
                w = plsc.bitcast(plsc.load_gather(tab_v, [idx]), jnp.int32)
                m0[b][pl.ds(off, 16)] = plsc.bitcast(
                    lax.shift_left(w, 16), jnp.float32)
                m1[b][pl.ds(off, 16)] = plsc.bitcast(
                    jnp.bitwise_and(w, mask_hi), jnp.float32)
            return 0
        return unpack_body

    def step(jj, i, b):
        base = wid * EW + i * MCH
        @pl.when(jj > 0)
        def _():
            pltpu.make_async_copy(m0[b], a0_sh.at[idx_d[b]], sem_s[b]).wait()
            pltpu.make_async_copy(m1[b], a1_sh.at[idx_d[b]], sem_s[b]).wait()
        li = pltpu.async_copy(src_hbm.at[pl.ds(base, MCH)], idx_s[b],
                              sem_i[b])
        lj = pltpu.async_copy(dst_hbm.at[pl.ds(base, MCH)], idx_d[b],
                              sem_i[b])
        li.wait()
        lj.wait()
        lax.fori_loop(0, MCH // 80, make_unpack(b), 0)
        pltpu.async_copy(m0[b], a0_sh.at[idx_d[b]], sem_s[b], add=True)
        pltpu.async_copy(m1[b], a1_sh.at[idx_d[b]], sem_s[b], add=True)

    def body(jj, _):
        step(jj, 2 * jj, 0)
        step(jj, 2 * jj + 1, 1)
        return 0

    lax.fori_loop(0, MITERS // 2, body, 0)
    for b in range(2):
        pltpu.make_async_copy(m0[b], a0_sh.at[idx_d[b]], sem_s[b]).wait()
        pltpu.make_async_copy(m1[b], a1_sh.at[idx_d[b]], sem_s[b]).wait()
    plsc.subcore_barrier()
    for c in range(4):
        qsl = pl.ds(sid * SUB + c * QS, QS)
        pltpu.sync_copy(a0_sh.at[qsl], m00.at[pl.ds(0, QS)])
        pltpu.sync_copy(m00.at[pl.ds(0, QS)], out_hbm.at[cid, 0, qsl])
        pltpu.sync_copy(a1_sh.at[qsl], m10.at[pl.ds(0, QS)])
        pltpu.sync_copy(m10.at[pl.ds(0, QS)], out_hbm.at[cid, 1, qsl])
    plsc.subcore_barrier()


_MSG_SCRATCH = [
    pltpu.VMEM((MCH,), jnp.int32),
    pltpu.VMEM((MCH,), jnp.int32),
    pltpu.VMEM((MCH,), jnp.int32),
    pltpu.VMEM((MCH,), jnp.int32),
    pltpu.VMEM((MCH,), jnp.float32),
    pltpu.VMEM((MCH,), jnp.float32),
    pltpu.VMEM((MCH,), jnp.float32),
    pltpu.VMEM((MCH,), jnp.float32),
    pltpu.VMEM((NP,), jnp.float32),
    pltpu.VMEM_SHARED((NP,), jnp.float32),
    pltpu.VMEM_SHARED((NP,), jnp.float32),
    pltpu.SemaphoreType.DMA,
    pltpu.SemaphoreType.DMA,
    pltpu.SemaphoreType.DMA,
    pltpu.SemaphoreType.DMA,
]


def _msg_layer1(src, dst, degp, x0p, x1p, wb1, zeros1):
    @functools.partial(
        pl.kernel,
        out_type=[
            jax.ShapeDtypeStruct((NC, 2, NP), jnp.float32),
            jax.ShapeDtypeStruct((NP,), jnp.float32),
        ],
        mesh=_mesh(),
        compiler_params=pltpu.CompilerParams(
            use_tc_tiling_on_sc=False, needs_layout_passes=False),
        scratch_types=_MSG_SCRATCH + [pltpu.VMEM((64,), jnp.float32)],
    )
    def k(src_hbm, dst_hbm, degp_hbm, x0_hbm, x1_hbm, wb_hbm, z_hbm,
          out_hbm, tpk_hbm,
          is0, is1, id0, id1, m00, m01, m10, m11, tab_v, a0_sh, a1_sh,
          sem_i0, sem_i1, sem_s0, sem_s1, wb_v):
        cid = lax.axis_index("c")
        sid = lax.axis_index("s")
        wid = cid * NS + sid
        pltpu.sync_copy(wb_hbm, wb_v)
        w00 = wb_v[pl.ds(0, 16)]
        w10 = wb_v[pl.ds(16, 16)]
        w01 = wb_v[pl.ds(32, 16)]
        w11 = wb_v[pl.ds(48, 16)]
        for off, ln in VCH:
            nsl = pl.ds(sid * SUB + off, ln)
            lsl = pl.ds(0, ln)
            pltpu.sync_copy(degp_hbm.at[0, nsl], m00.at[lsl])
            pltpu.sync_copy(degp_hbm.at[1, nsl], m01.at[lsl])
            pltpu.sync_copy(x0_hbm.at[nsl], m10.at[lsl])
            pltpu.sync_copy(x1_hbm.at[nsl], m11.at[lsl])

            def vbody(j, _):
                ds16 = pl.ds(j * 16, 16)
                deg = m00[ds16] + m01[ds16] + 1.0
                y = _rsqrt16(deg)
                x0 = m10[ds16]
                x1 = m11[ds16]
                g0 = y * (x0 * w00 + x1 * w10)
                g1 = y * (x0 * w01 + x1 * w11)
                m00[ds16] = plsc.bitcast(_pack16(g0, g1), jnp.float32)
                return 0

            lax.fori_loop(0, ln // 16, vbody, 0)
            pltpu.sync_copy(m00.at[lsl], a0_sh.at[nsl])
        plsc.subcore_barrier()
        pltpu.sync_copy(a0_sh, tab_v)

        @pl.when(cid == 0)
        def _():
            sl = pl.ds(sid * SUB, SUB)
            pltpu.sync_copy(tab_v.at[sl], tpk_hbm.at[sl])

        plsc.subcore_barrier()
        _msg_body(src_hbm, dst_hbm, z_hbm, out_hbm,
                  is0, is1, id0, id1, m00, m01, m10, m11, tab_v,
                  a0_sh, a1_sh, (sem_i0, sem_i1), (sem_s0, sem_s1),
                  cid, sid, wid)

    return k(src, dst, degp, x0p, x1p, wb1, zeros1)


def _msg_layer2(src, dst, tpk, zeros1):
    @functools.partial(
        pl.kernel,
        out_type=jax.ShapeDtypeStruct((NC, 2, NP), jnp.float32),
        mesh=_mesh(),
        compiler_params=pltpu.CompilerParams(
            use_tc_tiling_on_sc=False, needs_layout_passes=False),
        scratch_types=_MSG_SCRATCH,
    )
    def k(src_hbm, dst_hbm, tpk_hbm, z_hbm, out_hbm,
          is0, is1, id0, id1, m00, m01, m10, m11, tab_v, a0_sh, a1_sh,
          sem_i0, sem_i1, sem_s0, sem_s1):
        cid = lax.axis_index("c")
        sid = lax.axis_index("s")
        wid = cid * NS + sid
        pltpu.sync_copy(tpk_hbm, tab_v)
        _msg_body(src_hbm, dst_hbm, z_hbm, out_hbm,
                  is0, is1, id0, id1, m00, m01, m10, m11, tab_v,
                  a0_sh, a1_sh, (sem_i0, sem_i1), (sem_s0, sem_s1),
                  cid, sid, wid)

    return k(src, dst, tpk, zeros1)


def _tc_unpack(pk):
    w = lax.bitcast_convert_type(pk, jnp.int32)
    g0 = lax.bitcast_convert_type(lax.shift_left(w, 16), jnp.float32)
    g1 = lax.bitcast_convert_type(
        jnp.bitwise_and(w, jnp.int32(-65536)), jnp.float32)
    return g0, g1


def _tc_pack(g0, g1):
    b0 = lax.bitcast_convert_type(g0.astype(jnp.bfloat16), jnp.uint16)
    b1 = lax.bitcast_convert_type(g1.astype(jnp.bfloat16), jnp.uint16)
    w = (b1.astype(jnp.uint32) << 16) | b0.astype(jnp.uint32)
    return lax.bitcast_convert_type(w, jnp.float32)


def _tc_mid(degp, tpk1, Sa, Sb, W2, b1):
    def body(degp_ref, pk_ref, sa_ref, sb_ref, w_ref, b_ref, pk2_ref):
        deg = degp_ref[0:1, :] + degp_ref[1:2, :] + 1.0
        dinv = lax.rsqrt(deg)
        g0, g1 = _tc_unpack(pk_ref[...])
        o0 = dinv * (sa_ref[0:1, :] + sb_ref[0:1, :] + g0) + b_ref[0:1, :]
        o1 = dinv * (sa_ref[1:2, :] + sb_ref[1:2, :] + g1) + b_ref[1:2, :]
        g20 = dinv * (o0 * w_ref[0:1, 0:1] + o1 * w_ref[1:2, 0:1])
        g21 = dinv * (o0 * w_ref[0:1, 1:2] + o1 * w_ref[1:2, 1:2])
        pk2_ref[...] = _tc_pack(g20, g21)

    return pl.pallas_call(
        body,
        out_shape=jax.ShapeDtypeStruct((1, NP), jnp.float32),
    )(degp, tpk1, Sa, Sb, W2, b1)


def _tc_post(degp, tpk2, Sa, Sb, b2):
    def body(degp_ref, pk_ref, sa_ref, sb_ref, b_ref, out_ref):
        deg = degp_ref[0:1, :] + degp_ref[1:2, :] + 1.0
        dinv = lax.rsqrt(deg)
        g0, g1 = _tc_unpack(pk_ref[...])
        o0 = dinv * (sa_ref[0:1, :] + sb_ref[0:1, :] + g0) + b_ref[0:1, :]
        o1 = dinv * (sa_ref[1:2, :] + sb_ref[1:2, :] + g1) + b_ref[1:2, :]
        out_ref[...] = jnp.concatenate([o0, o1], axis=0)

    return pl.pallas_call(
        body,
        out_shape=jax.ShapeDtypeStruct((2, NP), jnp.float32),
    )(degp, tpk2, Sa, Sb, b2)


def kernel(x, edge_index, W1, b1, W2, b2):
    assert x.shape == (N, 2) and edge_index.shape == (2, E)
    src = edge_index[0]
    dst = edge_index[1]
    xT = jnp.pad(x, ((0, NP - N), (0, 0))).T
    b1c = jnp.broadcast_to(b1.reshape(2, 1), (2, NP))
    b2c = jnp.broadcast_to(b2.reshape(2, 1), (2, NP))
    zeros1 = jnp.zeros((NP,), jnp.float32)
    ones1 = jnp.ones((CH,), jnp.float32)
    wb1 = jnp.repeat(
        jnp.stack([W1[0, 0], W1[1, 0], W1[0, 1], W1[1, 1]]), 16)

    degp = _hist(dst, zeros1, ones1)
    S1p, tpk1 = _msg_layer1(src, dst, degp, xT[0], xT[1], wb1, zeros1)
    tpk2 = _tc_mid(degp, tpk1.reshape(1, NP), S1p[0], S1p[1], W2, b1c)
    S2p = _msg_layer2(src, dst, tpk2.reshape(NP), zeros1)
    out2T = _tc_post(degp, tpk2.reshape(1, NP), S2p[0], S2p[1], b2c)

    return out2T.T[:N]
